# baseline (speedup 1.0000x reference)
import jax
import jax.numpy as jnp
from jax import lax
from jax.experimental import pallas as pl
from jax.experimental.pallas import tpu as pltpu
from jax.experimental.pallas import tpu_sc as plsc

_B = 4096
_NN = 13
_NC = 26
_V = 100000
_D = 64
_T = 1 + _NN + _NC

_NW = 32
_BPW = _B // _NW
_CB = 8
_NCHUNK = _BPW // _CB

_NCAT = _CB * _NC


def _sc_body(xnum_hbm, xcat_hbm, w_hbm, bias_hbm, tab_hbm, cls_hbm, out_hbm,
             idx_v, asm0_v, asm1_v, asm2_v, xnum_v, w_v, bias_v, cls_v,
             gsem0, gsem1, gsem2, osem0, osem1, osem2):
    cid = lax.axis_index("c")
    sid = lax.axis_index("s")
    wid = sid * 2 + cid
    base = wid * _BPW

    pltpu.sync_copy(w_hbm, w_v)
    pltpu.sync_copy(bias_hbm, bias_v)
    pltpu.sync_copy(cls_hbm, cls_v)
    pltpu.sync_copy(xcat_hbm.at[pl.ds(base * _NC, _BPW * _NC)], idx_v)
    pltpu.sync_copy(xnum_hbm.at[pl.ds(base * 16, _BPW * 16)], xnum_v)

    def _fire(c, asm_v, gsem, osem, wait_prev):
        b0 = base + c * _CB
        dst = out_hbm.at[pl.ds(b0 * _T, _CB * _T)]

        @pl.when(wait_prev)
        def _():
            pltpu.make_async_copy(asm_v, dst, osem).wait()

        o = c * _NCAT
        for j in range(_NCAT // 16):
            vv = idx_v[pl.ds(o + j * 16, 16)]
            for i in range(16):
                p = j * 16 + i
                b, f = divmod(p, _NC)
                pltpu.async_copy(
                    tab_hbm.at[vv[i] + f * _V],
                    asm_v.at[b * _T + 1 + _NN + f], gsem)
        return dst

    def _finish(c, asm_v, gsem, osem, dst):
        clsk = [cls_v[pl.ds(k * 16, 16)] for k in range(_D // 16)]
        xvs = [xnum_v[pl.ds((c * _CB + b) * 16, 16)] for b in range(_CB)]
        for b in range(_CB):
            for k in range(_D // 16):
                asm_v[b * _T, pl.ds(k * 16, 16)] = clsk[k]
        for f in range(_NN):
            wk = [w_v[pl.ds(f * _D + k * 16, 16)] for k in range(_D // 16)]
            bk = [bias_v[pl.ds(f * _D + k * 16, 16)] for k in range(_D // 16)]
            for b in range(_CB):
                xs = xvs[b][f]
                for k in range(_D // 16):
                    s = pl.ds(k * 16, 16)
                    asm_v[b * _T + 1 + f, s] = xs * wk[k] + bk[k]

        pltpu.make_async_copy(
            tab_hbm.at[pl.ds(0, _NCAT)], asm_v.at[pl.ds(0, _NCAT)], gsem
        ).wait()

        pltpu.async_copy(asm_v, dst, osem)

    def _tri(p, _):
        dst0 = _fire(3 * p, asm0_v, gsem0, osem0, p >= 1)
        dst1 = _fire(3 * p + 1, asm1_v, gsem1, osem1, p >= 1)
        dst2 = _fire(3 * p + 2, asm2_v, gsem2, osem2, p >= 1)
        _finish(3 * p, asm0_v, gsem0, osem0, dst0)
        _finish(3 * p + 1, asm1_v, gsem1, osem1, dst1)
        _finish(3 * p + 2, asm2_v, gsem2, osem2, dst2)
        return 0

    lax.fori_loop(0, _NCHUNK // 3, _tri, 0)

    dstt = _fire(_NCHUNK - 1, asm0_v, gsem0, osem0, jnp.bool_(True))
    _finish(_NCHUNK - 1, asm0_v, gsem0, osem0, dstt)

    last = out_hbm.at[pl.ds(base * _T, _CB * _T)]
    pltpu.make_async_copy(asm0_v, last, osem0).wait()
    pltpu.make_async_copy(asm1_v, last, osem1).wait()
    pltpu.make_async_copy(asm2_v, last, osem2).wait()


@jax.jit
def _tokenize(x_num_flat, x_cat_flat, w_flat, bias_flat, tables_flat, cls_flat):
    mesh = plsc.VectorSubcoreMesh(core_axis_name="c", subcore_axis_name="s")
    kern = pl.kernel(
        _sc_body,
        out_type=jax.ShapeDtypeStruct((_B * _T, _D), jnp.float32),
        mesh=mesh,
        scratch_types=[
            pltpu.VMEM((_BPW * _NC,), jnp.int32),
            pltpu.VMEM((_CB * _T, _D), jnp.float32),
            pltpu.VMEM((_CB * _T, _D), jnp.float32),
            pltpu.VMEM((_CB * _T, _D), jnp.float32),
            pltpu.VMEM((_BPW * 16,), jnp.float32),
            pltpu.VMEM((_NN * _D,), jnp.float32),
            pltpu.VMEM((_NN * _D,), jnp.float32),
            pltpu.VMEM((_D,), jnp.float32),
            pltpu.SemaphoreType.DMA,
            pltpu.SemaphoreType.DMA,
            pltpu.SemaphoreType.DMA,
            pltpu.SemaphoreType.DMA,
            pltpu.SemaphoreType.DMA,
            pltpu.SemaphoreType.DMA,
        ],
        compiler_params=pltpu.CompilerParams(use_tc_tiling_on_sc=True),
    )
    return kern(x_num_flat, x_cat_flat, w_flat, bias_flat, tables_flat, cls_flat)


def kernel(x_num, x_cat, W_num, b_num, tables, cls_token):
    x_num_flat = jnp.pad(x_num, ((0, 0), (0, 16 - _NN))).reshape(_B * 16)
    x_cat_flat = x_cat.astype(jnp.int32).reshape(_B * _NC)
    tables_flat = tables.reshape(_NC * _V, _D)
    w_flat = W_num.reshape(_NN * _D)
    bias_flat = b_num.reshape(_NN * _D)
    cls_flat = cls_token.reshape(_D)
    out2 = _tokenize(x_num_flat, x_cat_flat, w_flat, bias_flat,
                     tables_flat, cls_flat)
    return out2.reshape(_B, _T, _D)


# speedup vs baseline: 1.0123x; 1.0123x over previous
import jax
import jax.numpy as jnp
from jax import lax
from jax.experimental import pallas as pl
from jax.experimental.pallas import tpu as pltpu
from jax.experimental.pallas import tpu_sc as plsc

_B = 4096
_NN = 13
_NC = 26
_V = 100000
_D = 64
_T = 1 + _NN + _NC

_NW = 32
_BPW = _B // _NW
_CB = 8
_NCHUNK = _BPW // _CB

_NCAT = _CB * _NC


def _sc_body(xnum_hbm, xcat_hbm, w_hbm, bias_hbm, tab_hbm, cls_hbm, out_hbm,
             idx_v, asm0_v, asm1_v, xnum_v, w_v, bias_v, cls_v,
             gsem0, gsem1, osem0, osem1):
    cid = lax.axis_index("c")
    sid = lax.axis_index("s")
    wid = sid * 2 + cid
    base = wid * _BPW

    pltpu.sync_copy(w_hbm, w_v)
    pltpu.sync_copy(bias_hbm, bias_v)
    pltpu.sync_copy(cls_hbm, cls_v)
    pltpu.sync_copy(xcat_hbm.at[pl.ds(base * _NC, _BPW * _NC)], idx_v)
    pltpu.sync_copy(xnum_hbm.at[pl.ds(base * 16, _BPW * 16)], xnum_v)

    def _fire(c, asm_v, gsem, osem, wait_prev):
        b0 = base + c * _CB
        dst = out_hbm.at[pl.ds(b0 * _T, _CB * _T)]

        @pl.when(wait_prev)
        def _():
            pltpu.make_async_copy(asm_v, dst, osem).wait()

        o = c * _NCAT
        for j in range(_NCAT // 16):
            vv = idx_v[pl.ds(o + j * 16, 16)]
            for i in range(16):
                p = j * 16 + i
                b, f = divmod(p, _NC)
                pltpu.async_copy(
                    tab_hbm.at[vv[i] + f * _V],
                    asm_v.at[b * _T + 1 + _NN + f], gsem)
        return dst

    def _finish(c, asm_v, gsem, osem, dst):
        clsk = [cls_v[pl.ds(k * 16, 16)] for k in range(_D // 16)]
        xvs = [xnum_v[pl.ds((c * _CB + b) * 16, 16)] for b in range(_CB)]
        for b in range(_CB):
            for k in range(_D // 16):
                asm_v[b * _T, pl.ds(k * 16, 16)] = clsk[k]
        for f in range(_NN):
            wk = [w_v[pl.ds(f * _D + k * 16, 16)] for k in range(_D // 16)]
            bk = [bias_v[pl.ds(f * _D + k * 16, 16)] for k in range(_D // 16)]
            for b in range(_CB):
                xs = xvs[b][f]
                for k in range(_D // 16):
                    s = pl.ds(k * 16, 16)
                    asm_v[b * _T + 1 + f, s] = xs * wk[k] + bk[k]

        pltpu.make_async_copy(
            tab_hbm.at[pl.ds(0, _NCAT)], asm_v.at[pl.ds(0, _NCAT)], gsem
        ).wait()

        pltpu.async_copy(asm_v, dst, osem)

    def _pair(p, _):
        dst0 = _fire(2 * p, asm0_v, gsem0, osem0, p >= 1)
        dst1 = _fire(2 * p + 1, asm1_v, gsem1, osem1, p >= 1)
        _finish(2 * p, asm0_v, gsem0, osem0, dst0)
        _finish(2 * p + 1, asm1_v, gsem1, osem1, dst1)
        return 0

    lax.fori_loop(0, _NCHUNK // 2, _pair, 0)

    last = out_hbm.at[pl.ds(base * _T, _CB * _T)]
    pltpu.make_async_copy(asm0_v, last, osem0).wait()
    pltpu.make_async_copy(asm1_v, last, osem1).wait()


@jax.jit
def _tokenize(x_num_flat, x_cat_flat, w_flat, bias_flat, tables_flat, cls_flat):
    mesh = plsc.VectorSubcoreMesh(core_axis_name="c", subcore_axis_name="s")
    kern = pl.kernel(
        _sc_body,
        out_type=jax.ShapeDtypeStruct((_B * _T, _D), jnp.float32),
        mesh=mesh,
        scratch_types=[
            pltpu.VMEM((_BPW * _NC,), jnp.int32),
            pltpu.VMEM((_CB * _T, _D), jnp.float32),
            pltpu.VMEM((_CB * _T, _D), jnp.float32),
            pltpu.VMEM((_BPW * 16,), jnp.float32),
            pltpu.VMEM((_NN * _D,), jnp.float32),
            pltpu.VMEM((_NN * _D,), jnp.float32),
            pltpu.VMEM((_D,), jnp.float32),
            pltpu.SemaphoreType.DMA,
            pltpu.SemaphoreType.DMA,
            pltpu.SemaphoreType.DMA,
            pltpu.SemaphoreType.DMA,
        ],
        compiler_params=pltpu.CompilerParams(use_tc_tiling_on_sc=True),
    )
    return kern(x_num_flat, x_cat_flat, w_flat, bias_flat, tables_flat, cls_flat)


def kernel(x_num, x_cat, W_num, b_num, tables, cls_token):
    x_num_flat = jnp.pad(x_num, ((0, 0), (0, 16 - _NN))).reshape(_B * 16)
    x_cat_flat = x_cat.astype(jnp.int32).reshape(_B * _NC)
    tables_flat = tables.reshape(_NC * _V, _D)
    w_flat = W_num.reshape(_NN * _D)
    bias_flat = b_num.reshape(_NN * _D)
    cls_flat = cls_token.reshape(_D)
    out2 = _tokenize(x_num_flat, x_cat_flat, w_flat, bias_flat,
                     tables_flat, cls_flat)
    return out2.reshape(_B, _T, _D)
